# baseline (speedup 1.0000x reference)
import functools

import jax
import jax.numpy as jnp
from jax.experimental import pallas as pl
from jax.experimental.pallas import tpu as pltpu

N = 50; T = 100; B = 16; C = 32; H = 512; DH = 150; OUT = 50
NP = 64
NC = N * C
G3 = 3 * H
DP = 256
G3D = 3 * DP
OUTP = 128
BT = B * T
CHUNK = 160
NCHUNK = BT // CHUNK
TCH = CHUNK // B


def _prep_kernel(emb_ref, ew_ref, ev_ref, g1_ref):
    emb = emb_ref[...]
    sq = jnp.sum(emb * emb, axis=1, keepdims=True)
    norm = jnp.maximum(jnp.sqrt(sq), 1e-8)
    wn = emb / norm
    a = jax.lax.dot_general(wn, wn, (((1,), (1,)), ((), ())),
                            preferred_element_type=jnp.float32)
    ii = jax.lax.broadcasted_iota(jnp.int32, (NP, NP), 0)
    jj = jax.lax.broadcasted_iota(jnp.int32, (NP, NP), 1)
    a = jnp.where(ii == jj, 0.0, a)
    a = jnp.maximum(a, 0.0)
    deg = jnp.sum(a, axis=0, keepdims=True)
    dis = jnp.where(deg > 0, jax.lax.rsqrt(deg), 0.0)
    m = a * dis * jnp.transpose(dis)
    g1_ref[...] = jnp.dot(m, ew_ref[...],
                          preferred_element_type=jnp.float32) + ev_ref[...]


def _gemm_kernel(x_ref, g1_ref, brow_ref, wih_ref, bih_ref, gi_ref):
    y = jnp.dot(x_ref[...], g1_ref[...], preferred_element_type=jnp.float32)
    y = y + brow_ref[...]
    g = 0.5 * y * (1.0 + jax.lax.erf(y * 0.7071067811865476))
    gi_ref[...] = jnp.dot(g.astype(jnp.bfloat16), wih_ref[...],
                          preferred_element_type=jnp.float32) + bih_ref[...]


def _enc_kernel(gi_ref, whh_ref, bhh_ref, hend_ref, h_ref):
    pid = pl.program_id(0)

    @pl.when(pid == 0)
    def _():
        h_ref[...] = jnp.zeros((B, H), jnp.float32)

    def step(i, h):
        gi = gi_ref[pl.ds(i * B, B), :]
        gh = jnp.dot(h, whh_ref[...],
                     preferred_element_type=jnp.float32) + bhh_ref[...]
        r = jax.nn.sigmoid(gi[:, :H] + gh[:, :H])
        z = jax.nn.sigmoid(gi[:, H:2 * H] + gh[:, H:2 * H])
        n = jnp.tanh(gi[:, 2 * H:] + r * gh[:, 2 * H:])
        return (1.0 - z) * n + z * h

    h = jax.lax.fori_loop(0, TCH, step, h_ref[...])
    h_ref[...] = h

    @pl.when(pid == NCHUNK - 1)
    def _():
        hend_ref[...] = h


def _dec_kernel(rep_ref, dwih_ref, dbih_ref, dwhh_ref, dbhh_ref, fcw_ref,
                fcb_ref, out_ref, gid_ref):
    gid_ref[...] = jnp.dot(rep_ref[...], dwih_ref[...],
                           preferred_element_type=jnp.float32) + dbih_ref[...]

    def dstep(t, hd):
        gi = gid_ref[pl.ds(t * B, B), :]
        ghd = jnp.dot(hd, dwhh_ref[...],
                      preferred_element_type=jnp.float32) + dbhh_ref[...]
        r = jax.nn.sigmoid(gi[:, :DP] + ghd[:, :DP])
        z = jax.nn.sigmoid(gi[:, DP:2 * DP] + ghd[:, DP:2 * DP])
        n = jnp.tanh(gi[:, 2 * DP:] + r * ghd[:, 2 * DP:])
        hd = (1.0 - z) * n + z * hd
        out_ref[pl.ds(t * B, B), :] = jnp.dot(
            hd, fcw_ref[...], preferred_element_type=jnp.float32) + fcb_ref[...]
        return hd

    jax.lax.fori_loop(0, T, dstep, jnp.zeros((B, DP), jnp.float32))


def _pad2(x, r, c):
    return jnp.pad(x, ((0, r - x.shape[0]), (0, c - x.shape[1])))


@jax.jit
def kernel(window, emb_W, arma_w, arma_v, arma_b, gru_Wih, gru_Whh, gru_bih,
           gru_bhh, dec_Wih, dec_Whh, dec_bih, dec_bhh, fc_W, fc_b):
    f32 = jnp.float32
    xtb = jnp.transpose(window, (1, 0, 2)).reshape(BT, N)
    x_pad = _pad2(xtb, BT, NP)
    emb_pad = _pad2(emb_W, NP, 128)
    eye = jnp.eye(N, dtype=f32)
    ew = _pad2((eye[:, :, None] * arma_w[0][None, None, :]).reshape(N, NC), NP, NC)
    ev = _pad2((eye[:, :, None] * arma_v[0][None, None, :]).reshape(N, NC), NP, NC)
    brow = jnp.tile(arma_b, N)[None, :]
    wihT = gru_Wih.T.astype(jnp.bfloat16)
    bih = gru_bih[None, :]
    whhT = gru_Whh.T
    bhh = gru_bhh[None, :]
    dwihT = jnp.concatenate(
        [_pad2(dec_Wih[g * DH:(g + 1) * DH, :].T, H, DP) for g in range(3)],
        axis=1)
    dbih = jnp.concatenate(
        [jnp.pad(dec_bih[g * DH:(g + 1) * DH], (0, DP - DH)) for g in range(3)]
    )[None, :]
    dwhhT = jnp.concatenate(
        [_pad2(dec_Whh[g * DH:(g + 1) * DH, :].T, DP, DP) for g in range(3)],
        axis=1)
    dbhh = jnp.concatenate(
        [jnp.pad(dec_bhh[g * DH:(g + 1) * DH], (0, DP - DH)) for g in range(3)]
    )[None, :]
    fcwT = _pad2(fc_W.T, DP, OUTP)
    fcb = jnp.pad(fc_b, (0, OUTP - OUT))[None, :]

    g1 = pl.pallas_call(
        _prep_kernel,
        out_shape=jax.ShapeDtypeStruct((NP, NC), f32),
    )(emb_pad, ew, ev)

    gi = pl.pallas_call(
        _gemm_kernel,
        grid=(NCHUNK,),
        in_specs=[
            pl.BlockSpec((CHUNK, NP), lambda i: (i, 0)),
            pl.BlockSpec((NP, NC), lambda i: (0, 0)),
            pl.BlockSpec((1, NC), lambda i: (0, 0)),
            pl.BlockSpec((NC, G3), lambda i: (0, 0)),
            pl.BlockSpec((1, G3), lambda i: (0, 0)),
        ],
        out_specs=pl.BlockSpec((CHUNK, G3), lambda i: (i, 0)),
        out_shape=jax.ShapeDtypeStruct((BT, G3), f32),
    )(x_pad, g1, brow, wihT, bih)

    h_end = pl.pallas_call(
        _enc_kernel,
        grid=(NCHUNK,),
        in_specs=[
            pl.BlockSpec((CHUNK, G3), lambda i: (i, 0)),
            pl.BlockSpec((H, G3), lambda i: (0, 0)),
            pl.BlockSpec((1, G3), lambda i: (0, 0)),
        ],
        out_specs=pl.BlockSpec((B, H), lambda i: (0, 0)),
        out_shape=jax.ShapeDtypeStruct((B, H), f32),
        scratch_shapes=[pltpu.VMEM((B, H), f32)],
    )(gi, whhT, bhh)

    rep = jnp.repeat(h_end, T, axis=1).reshape(B, T, H)
    rep_tb = rep.transpose(1, 0, 2).reshape(BT, H)

    out = pl.pallas_call(
        _dec_kernel,
        in_specs=[
            pl.BlockSpec((BT, H), lambda: (0, 0)),
            pl.BlockSpec((H, G3D), lambda: (0, 0)),
            pl.BlockSpec((1, G3D), lambda: (0, 0)),
            pl.BlockSpec((DP, G3D), lambda: (0, 0)),
            pl.BlockSpec((1, G3D), lambda: (0, 0)),
            pl.BlockSpec((DP, OUTP), lambda: (0, 0)),
            pl.BlockSpec((1, OUTP), lambda: (0, 0)),
        ],
        out_specs=pl.BlockSpec((BT, OUTP), lambda: (0, 0)),
        out_shape=jax.ShapeDtypeStruct((BT, OUTP), f32),
        scratch_shapes=[pltpu.VMEM((BT, G3D), f32)],
    )(rep_tb, dwihT, dbih, dwhhT, dbhh, fcwT, fcb)

    return out[:, :OUT].reshape(T, B, OUT).transpose(1, 0, 2)


# speedup vs baseline: 1.1337x; 1.1337x over previous
import functools

import jax
import jax.numpy as jnp
from jax.experimental import pallas as pl
from jax.experimental.pallas import tpu as pltpu

N = 50; T = 100; B = 16; C = 32; H = 512; DH = 150; OUT = 50
NP = 64
NC = N * C
G3 = 3 * H
DP = 256
G3D = 3 * DP
OUTP = 128
BT = B * T
CHUNK = 160
NCHUNK = BT // CHUNK
TCH = CHUNK // B


def _prep_kernel(emb_ref, ew_ref, ev_ref, g1_ref):
    emb = emb_ref[...]
    sq = jnp.sum(emb * emb, axis=1, keepdims=True)
    norm = jnp.maximum(jnp.sqrt(sq), 1e-8)
    wn = emb / norm
    a = jax.lax.dot_general(wn, wn, (((1,), (1,)), ((), ())),
                            preferred_element_type=jnp.float32)
    ii = jax.lax.broadcasted_iota(jnp.int32, (NP, NP), 0)
    jj = jax.lax.broadcasted_iota(jnp.int32, (NP, NP), 1)
    a = jnp.where(ii == jj, 0.0, a)
    a = jnp.maximum(a, 0.0)
    deg = jnp.sum(a, axis=0, keepdims=True)
    dis = jnp.where(deg > 0, jax.lax.rsqrt(deg), 0.0)
    m = a * dis * jnp.transpose(dis)
    g1_ref[...] = jnp.dot(m, ew_ref[...],
                          preferred_element_type=jnp.float32) + ev_ref[...]


def _gemm_kernel(x_ref, g1_ref, brow_ref, wih_ref, bih_ref, gi_ref):
    y = jnp.dot(x_ref[...], g1_ref[...], preferred_element_type=jnp.float32)
    y = y + brow_ref[...]
    g = 0.5 * y * (1.0 + jax.lax.erf(y * 0.7071067811865476))
    gi_ref[...] = jnp.dot(g, wih_ref[...],
                          preferred_element_type=jnp.float32) + bih_ref[...]


def _enc_kernel(gi_ref, whh_ref, bhh_ref, hend_ref, h_ref):
    pid = pl.program_id(0)

    @pl.when(pid == 0)
    def _():
        h_ref[...] = jnp.zeros((B, H), jnp.float32)

    def step(i, h):
        gi = gi_ref[pl.ds(i * B, B), :]
        gh = jnp.dot(h, whh_ref[...],
                     preferred_element_type=jnp.float32) + bhh_ref[...]
        r = jax.nn.sigmoid(gi[:, :H] + gh[:, :H])
        z = jax.nn.sigmoid(gi[:, H:2 * H] + gh[:, H:2 * H])
        n = jnp.tanh(gi[:, 2 * H:] + r * gh[:, 2 * H:])
        return (1.0 - z) * n + z * h

    h = jax.lax.fori_loop(0, TCH, step, h_ref[...])
    h_ref[...] = h

    @pl.when(pid == NCHUNK - 1)
    def _():
        hend_ref[...] = h


def _dec_kernel(rep_ref, dwih_ref, dbih_ref, dwhh_ref, dbhh_ref, fcw_ref,
                fcb_ref, out_ref, gid_ref, hs_ref):
    gid_ref[...] = jnp.dot(rep_ref[...], dwih_ref[...],
                           preferred_element_type=jnp.float32) + dbih_ref[...]

    def dstep(t, hd):
        gi = gid_ref[pl.ds(t * B, B), :]
        ghd = jnp.dot(hd, dwhh_ref[...],
                      preferred_element_type=jnp.float32) + dbhh_ref[...]
        r = jax.nn.sigmoid(gi[:, :DP] + ghd[:, :DP])
        z = jax.nn.sigmoid(gi[:, DP:2 * DP] + ghd[:, DP:2 * DP])
        n = jnp.tanh(gi[:, 2 * DP:] + r * ghd[:, 2 * DP:])
        hd = (1.0 - z) * n + z * hd
        hs_ref[pl.ds(t * B, B), :] = hd
        return hd

    jax.lax.fori_loop(0, T, dstep, jnp.zeros((B, DP), jnp.float32))
    out_ref[...] = jnp.dot(hs_ref[...], fcw_ref[...],
                           preferred_element_type=jnp.float32) + fcb_ref[...]


def _pad2(x, r, c):
    return jnp.pad(x, ((0, r - x.shape[0]), (0, c - x.shape[1])))


@jax.jit
def kernel(window, emb_W, arma_w, arma_v, arma_b, gru_Wih, gru_Whh, gru_bih,
           gru_bhh, dec_Wih, dec_Whh, dec_bih, dec_bhh, fc_W, fc_b):
    f32 = jnp.float32
    xtb = jnp.transpose(window, (1, 0, 2)).reshape(BT, N)
    x_pad = _pad2(xtb, BT, NP)
    emb_pad = _pad2(emb_W, NP, 128)
    eye = jnp.eye(N, dtype=f32)
    ew = _pad2((eye[:, :, None] * arma_w[0][None, None, :]).reshape(N, NC), NP, NC)
    ev = _pad2((eye[:, :, None] * arma_v[0][None, None, :]).reshape(N, NC), NP, NC)
    brow = jnp.tile(arma_b, N)[None, :]
    wihT = gru_Wih.T
    bih = gru_bih[None, :]
    whhT = gru_Whh.T
    bhh = gru_bhh[None, :]
    dwihT = jnp.concatenate(
        [_pad2(dec_Wih[g * DH:(g + 1) * DH, :].T, H, DP) for g in range(3)],
        axis=1)
    dbih = jnp.concatenate(
        [jnp.pad(dec_bih[g * DH:(g + 1) * DH], (0, DP - DH)) for g in range(3)]
    )[None, :]
    dwhhT = jnp.concatenate(
        [_pad2(dec_Whh[g * DH:(g + 1) * DH, :].T, DP, DP) for g in range(3)],
        axis=1)
    dbhh = jnp.concatenate(
        [jnp.pad(dec_bhh[g * DH:(g + 1) * DH], (0, DP - DH)) for g in range(3)]
    )[None, :]
    fcwT = _pad2(fc_W.T, DP, OUTP)
    fcb = jnp.pad(fc_b, (0, OUTP - OUT))[None, :]

    g1 = pl.pallas_call(
        _prep_kernel,
        out_shape=jax.ShapeDtypeStruct((NP, NC), f32),
    )(emb_pad, ew, ev)

    gi = pl.pallas_call(
        _gemm_kernel,
        grid=(NCHUNK,),
        in_specs=[
            pl.BlockSpec((CHUNK, NP), lambda i: (i, 0)),
            pl.BlockSpec((NP, NC), lambda i: (0, 0)),
            pl.BlockSpec((1, NC), lambda i: (0, 0)),
            pl.BlockSpec((NC, G3), lambda i: (0, 0)),
            pl.BlockSpec((1, G3), lambda i: (0, 0)),
        ],
        out_specs=pl.BlockSpec((CHUNK, G3), lambda i: (i, 0)),
        out_shape=jax.ShapeDtypeStruct((BT, G3), f32),
    )(x_pad, g1, brow, wihT, bih)

    h_end = pl.pallas_call(
        _enc_kernel,
        grid=(NCHUNK,),
        in_specs=[
            pl.BlockSpec((CHUNK, G3), lambda i: (i, 0)),
            pl.BlockSpec((H, G3), lambda i: (0, 0)),
            pl.BlockSpec((1, G3), lambda i: (0, 0)),
        ],
        out_specs=pl.BlockSpec((B, H), lambda i: (0, 0)),
        out_shape=jax.ShapeDtypeStruct((B, H), f32),
        scratch_shapes=[pltpu.VMEM((B, H), f32)],
    )(gi, whhT, bhh)

    rep = jnp.repeat(h_end, T, axis=1).reshape(B, T, H)
    rep_tb = rep.transpose(1, 0, 2).reshape(BT, H)

    out = pl.pallas_call(
        _dec_kernel,
        in_specs=[
            pl.BlockSpec((BT, H), lambda: (0, 0)),
            pl.BlockSpec((H, G3D), lambda: (0, 0)),
            pl.BlockSpec((1, G3D), lambda: (0, 0)),
            pl.BlockSpec((DP, G3D), lambda: (0, 0)),
            pl.BlockSpec((1, G3D), lambda: (0, 0)),
            pl.BlockSpec((DP, OUTP), lambda: (0, 0)),
            pl.BlockSpec((1, OUTP), lambda: (0, 0)),
        ],
        out_specs=pl.BlockSpec((BT, OUTP), lambda: (0, 0)),
        out_shape=jax.ShapeDtypeStruct((BT, OUTP), f32),
        scratch_shapes=[pltpu.VMEM((BT, G3D), f32), pltpu.VMEM((BT, DP), f32)],
    )(rep_tb, dwihT, dbih, dwhhT, dbhh, fcwT, fcb)

    return out[:, :OUT].reshape(T, B, OUT).transpose(1, 0, 2)


# speedup vs baseline: 1.1568x; 1.0203x over previous
import functools

import jax
import jax.numpy as jnp
from jax.experimental import pallas as pl
from jax.experimental.pallas import tpu as pltpu

N = 50; T = 100; B = 16; C = 32; H = 512; DH = 150; OUT = 50
NP = 64
NC = N * C
G3 = 3 * H
DP = 256
G3D = 3 * DP
OUTP = 128
BT = B * T
CHUNK = 160
NCHUNK = BT // CHUNK
TCH = CHUNK // B


def _prep_kernel(emb_ref, ew_ref, ev_ref, g1_ref):
    emb = emb_ref[...]
    sq = jnp.sum(emb * emb, axis=1, keepdims=True)
    norm = jnp.maximum(jnp.sqrt(sq), 1e-8)
    wn = emb / norm
    a = jax.lax.dot_general(wn, wn, (((1,), (1,)), ((), ())),
                            preferred_element_type=jnp.float32)
    ii = jax.lax.broadcasted_iota(jnp.int32, (NP, NP), 0)
    jj = jax.lax.broadcasted_iota(jnp.int32, (NP, NP), 1)
    a = jnp.where(ii == jj, 0.0, a)
    a = jnp.maximum(a, 0.0)
    deg = jnp.sum(a, axis=0, keepdims=True)
    dis = jnp.where(deg > 0, jax.lax.rsqrt(deg), 0.0)
    m = a * dis * jnp.transpose(dis)
    g1_ref[...] = jnp.dot(m, ew_ref[...],
                          preferred_element_type=jnp.float32) + ev_ref[...]


def _gemm_kernel(x_ref, g1_ref, brow_ref, wih_ref, bih_ref, gi_ref):
    y = jnp.dot(x_ref[...], g1_ref[...], preferred_element_type=jnp.float32)
    y = y + brow_ref[...]
    g = 0.5 * y * (1.0 + jax.lax.erf(y * 0.7071067811865476))
    gi_ref[...] = jnp.dot(g, wih_ref[...],
                          preferred_element_type=jnp.float32) + bih_ref[...]


def _enc_kernel(gi_ref, whh_ref, bhh_ref, hend_ref, h_ref):
    pid = pl.program_id(0)

    @pl.when(pid == 0)
    def _():
        h_ref[...] = jnp.zeros((B, H), jnp.float32)

    def step(i, h):
        gi = gi_ref[pl.ds(i * B, B), :]
        gh = jnp.dot(h.astype(jnp.bfloat16), whh_ref[...],
                     preferred_element_type=jnp.float32) + bhh_ref[...]
        r = jax.nn.sigmoid(gi[:, :H] + gh[:, :H])
        z = jax.nn.sigmoid(gi[:, H:2 * H] + gh[:, H:2 * H])
        n = jnp.tanh(gi[:, 2 * H:] + r * gh[:, 2 * H:])
        return (1.0 - z) * n + z * h

    h = jax.lax.fori_loop(0, TCH, step, h_ref[...])
    h_ref[...] = h

    @pl.when(pid == NCHUNK - 1)
    def _():
        hend_ref[...] = h


def _dec_kernel(rep_ref, dwih_ref, dbih_ref, dwhh_ref, dbhh_ref, fcw_ref,
                fcb_ref, out_ref, gid_ref, hs_ref):
    gid_ref[...] = jnp.dot(rep_ref[...], dwih_ref[...],
                           preferred_element_type=jnp.float32) + dbih_ref[...]

    def dstep(t, hd):
        gi = gid_ref[pl.ds(t * B, B), :]
        ghd = jnp.dot(hd.astype(jnp.bfloat16), dwhh_ref[...],
                      preferred_element_type=jnp.float32) + dbhh_ref[...]
        r = jax.nn.sigmoid(gi[:, :DP] + ghd[:, :DP])
        z = jax.nn.sigmoid(gi[:, DP:2 * DP] + ghd[:, DP:2 * DP])
        n = jnp.tanh(gi[:, 2 * DP:] + r * ghd[:, 2 * DP:])
        hd = (1.0 - z) * n + z * hd
        hs_ref[pl.ds(t * B, B), :] = hd
        return hd

    jax.lax.fori_loop(0, T, dstep, jnp.zeros((B, DP), jnp.float32))
    out_ref[...] = jnp.dot(hs_ref[...], fcw_ref[...],
                           preferred_element_type=jnp.float32) + fcb_ref[...]


def _pad2(x, r, c):
    return jnp.pad(x, ((0, r - x.shape[0]), (0, c - x.shape[1])))


@jax.jit
def kernel(window, emb_W, arma_w, arma_v, arma_b, gru_Wih, gru_Whh, gru_bih,
           gru_bhh, dec_Wih, dec_Whh, dec_bih, dec_bhh, fc_W, fc_b):
    f32 = jnp.float32
    xtb = jnp.transpose(window, (1, 0, 2)).reshape(BT, N)
    x_pad = _pad2(xtb, BT, NP)
    emb_pad = _pad2(emb_W, NP, 128)
    eye = jnp.eye(N, dtype=f32)
    ew = _pad2((eye[:, :, None] * arma_w[0][None, None, :]).reshape(N, NC), NP, NC)
    ev = _pad2((eye[:, :, None] * arma_v[0][None, None, :]).reshape(N, NC), NP, NC)
    brow = jnp.tile(arma_b, N)[None, :]
    wihT = gru_Wih.T
    bih = gru_bih[None, :]
    whhT = gru_Whh.T.astype(jnp.bfloat16)
    bhh = gru_bhh[None, :]
    dwihT = jnp.concatenate(
        [_pad2(dec_Wih[g * DH:(g + 1) * DH, :].T, H, DP) for g in range(3)],
        axis=1)
    dbih = jnp.concatenate(
        [jnp.pad(dec_bih[g * DH:(g + 1) * DH], (0, DP - DH)) for g in range(3)]
    )[None, :]
    dwhhT = jnp.concatenate(
        [_pad2(dec_Whh[g * DH:(g + 1) * DH, :].T, DP, DP) for g in range(3)],
        axis=1).astype(jnp.bfloat16)
    dbhh = jnp.concatenate(
        [jnp.pad(dec_bhh[g * DH:(g + 1) * DH], (0, DP - DH)) for g in range(3)]
    )[None, :]
    fcwT = _pad2(fc_W.T, DP, OUTP)
    fcb = jnp.pad(fc_b, (0, OUTP - OUT))[None, :]

    g1 = pl.pallas_call(
        _prep_kernel,
        out_shape=jax.ShapeDtypeStruct((NP, NC), f32),
    )(emb_pad, ew, ev)

    gi = pl.pallas_call(
        _gemm_kernel,
        grid=(NCHUNK,),
        in_specs=[
            pl.BlockSpec((CHUNK, NP), lambda i: (i, 0)),
            pl.BlockSpec((NP, NC), lambda i: (0, 0)),
            pl.BlockSpec((1, NC), lambda i: (0, 0)),
            pl.BlockSpec((NC, G3), lambda i: (0, 0)),
            pl.BlockSpec((1, G3), lambda i: (0, 0)),
        ],
        out_specs=pl.BlockSpec((CHUNK, G3), lambda i: (i, 0)),
        out_shape=jax.ShapeDtypeStruct((BT, G3), f32),
    )(x_pad, g1, brow, wihT, bih)

    h_end = pl.pallas_call(
        _enc_kernel,
        grid=(NCHUNK,),
        in_specs=[
            pl.BlockSpec((CHUNK, G3), lambda i: (i, 0)),
            pl.BlockSpec((H, G3), lambda i: (0, 0)),
            pl.BlockSpec((1, G3), lambda i: (0, 0)),
        ],
        out_specs=pl.BlockSpec((B, H), lambda i: (0, 0)),
        out_shape=jax.ShapeDtypeStruct((B, H), f32),
        scratch_shapes=[pltpu.VMEM((B, H), f32)],
    )(gi, whhT, bhh)

    rep = jnp.repeat(h_end, T, axis=1).reshape(B, T, H)
    rep_tb = rep.transpose(1, 0, 2).reshape(BT, H)

    out = pl.pallas_call(
        _dec_kernel,
        in_specs=[
            pl.BlockSpec((BT, H), lambda: (0, 0)),
            pl.BlockSpec((H, G3D), lambda: (0, 0)),
            pl.BlockSpec((1, G3D), lambda: (0, 0)),
            pl.BlockSpec((DP, G3D), lambda: (0, 0)),
            pl.BlockSpec((1, G3D), lambda: (0, 0)),
            pl.BlockSpec((DP, OUTP), lambda: (0, 0)),
            pl.BlockSpec((1, OUTP), lambda: (0, 0)),
        ],
        out_specs=pl.BlockSpec((BT, OUTP), lambda: (0, 0)),
        out_shape=jax.ShapeDtypeStruct((BT, OUTP), f32),
        scratch_shapes=[pltpu.VMEM((BT, G3D), f32), pltpu.VMEM((BT, DP), f32)],
    )(rep_tb, dwihT, dbih, dwhhT, dbhh, fcwT, fcb)

    return out[:, :OUT].reshape(T, B, OUT).transpose(1, 0, 2)


# speedup vs baseline: 1.1569x; 1.0001x over previous
import functools

import jax
import jax.numpy as jnp
from jax.experimental import pallas as pl
from jax.experimental.pallas import tpu as pltpu

N = 50; T = 100; B = 16; C = 32; H = 512; DH = 150; OUT = 50
NP = 64
NC = N * C
G3 = 3 * H
DP = 256
G3D = 3 * DP
OUTP = 128
BT = B * T
CHUNK = 160
NCHUNK = BT // CHUNK
TCH = CHUNK // B


def _prep_kernel(emb_ref, ew_ref, ev_ref, g1_ref):
    emb = emb_ref[...]
    sq = jnp.sum(emb * emb, axis=1, keepdims=True)
    norm = jnp.maximum(jnp.sqrt(sq), 1e-8)
    wn = emb / norm
    a = jax.lax.dot_general(wn, wn, (((1,), (1,)), ((), ())),
                            preferred_element_type=jnp.float32)
    ii = jax.lax.broadcasted_iota(jnp.int32, (NP, NP), 0)
    jj = jax.lax.broadcasted_iota(jnp.int32, (NP, NP), 1)
    a = jnp.where(ii == jj, 0.0, a)
    a = jnp.maximum(a, 0.0)
    deg = jnp.sum(a, axis=0, keepdims=True)
    dis = jnp.where(deg > 0, jax.lax.rsqrt(deg), 0.0)
    m = a * dis * jnp.transpose(dis)
    g1_ref[...] = jnp.dot(m, ew_ref[...],
                          preferred_element_type=jnp.float32) + ev_ref[...]


def _gemm_kernel(x_ref, g1_ref, brow_ref, wih_ref, bih_ref, gi_ref):
    y = jnp.dot(x_ref[...], g1_ref[...], preferred_element_type=jnp.float32)
    y = y + brow_ref[...]
    g = 0.5 * y * (1.0 + jax.lax.erf(y * 0.7071067811865476))
    gi_ref[...] = jnp.dot(g, wih_ref[...],
                          preferred_element_type=jnp.float32) + bih_ref[...]


def _enc_kernel(gi_ref, whh_ref, bhh_ref, hend_ref, h_ref):
    pid = pl.program_id(0)

    @pl.when(pid == 0)
    def _():
        h_ref[...] = jnp.zeros((B, H), jnp.float32)

    def step(i, h):
        gi = gi_ref[i]
        gh = jnp.dot(h.astype(jnp.bfloat16), whh_ref[...],
                     preferred_element_type=jnp.float32) + bhh_ref[...]
        r = jax.nn.sigmoid(gi[:, :H] + gh[:, :H])
        z = jax.nn.sigmoid(gi[:, H:2 * H] + gh[:, H:2 * H])
        n = jnp.tanh(gi[:, 2 * H:] + r * gh[:, 2 * H:])
        return (1.0 - z) * n + z * h

    h = jax.lax.fori_loop(0, TCH, step, h_ref[...])
    h_ref[...] = h

    @pl.when(pid == NCHUNK - 1)
    def _():
        hend_ref[...] = h


def _dec_kernel(rep_ref, dwih_ref, dbih_ref, dwhh_ref, dbhh_ref, fcw_ref,
                fcb_ref, out_ref, gid_ref, hs_ref):
    gid_ref[...] = (jnp.dot(rep_ref[...], dwih_ref[...],
                            preferred_element_type=jnp.float32)
                    + dbih_ref[...]).reshape(T, B, G3D)

    def dstep(t, hd):
        gi = gid_ref[t]
        ghd = jnp.dot(hd.astype(jnp.bfloat16), dwhh_ref[...],
                      preferred_element_type=jnp.float32) + dbhh_ref[...]
        r = jax.nn.sigmoid(gi[:, :DP] + ghd[:, :DP])
        z = jax.nn.sigmoid(gi[:, DP:2 * DP] + ghd[:, DP:2 * DP])
        n = jnp.tanh(gi[:, 2 * DP:] + r * ghd[:, 2 * DP:])
        hd = (1.0 - z) * n + z * hd
        hs_ref[t] = hd
        return hd

    jax.lax.fori_loop(0, T, dstep, jnp.zeros((B, DP), jnp.float32))
    out_ref[...] = jnp.dot(hs_ref[...].reshape(BT, DP), fcw_ref[...],
                           preferred_element_type=jnp.float32) + fcb_ref[...]


def _pad2(x, r, c):
    return jnp.pad(x, ((0, r - x.shape[0]), (0, c - x.shape[1])))


@jax.jit
def kernel(window, emb_W, arma_w, arma_v, arma_b, gru_Wih, gru_Whh, gru_bih,
           gru_bhh, dec_Wih, dec_Whh, dec_bih, dec_bhh, fc_W, fc_b):
    f32 = jnp.float32
    xtb = jnp.transpose(window, (1, 0, 2)).reshape(BT, N)
    x_pad = _pad2(xtb, BT, NP)
    emb_pad = _pad2(emb_W, NP, 128)
    eye = jnp.eye(N, dtype=f32)
    ew = _pad2((eye[:, :, None] * arma_w[0][None, None, :]).reshape(N, NC), NP, NC)
    ev = _pad2((eye[:, :, None] * arma_v[0][None, None, :]).reshape(N, NC), NP, NC)
    brow = jnp.tile(arma_b, N)[None, :]
    wihT = gru_Wih.T
    bih = gru_bih[None, :]
    whhT = gru_Whh.T.astype(jnp.bfloat16)
    bhh = gru_bhh[None, :]
    dwihT = jnp.concatenate(
        [_pad2(dec_Wih[g * DH:(g + 1) * DH, :].T, H, DP) for g in range(3)],
        axis=1)
    dbih = jnp.concatenate(
        [jnp.pad(dec_bih[g * DH:(g + 1) * DH], (0, DP - DH)) for g in range(3)]
    )[None, :]
    dwhhT = jnp.concatenate(
        [_pad2(dec_Whh[g * DH:(g + 1) * DH, :].T, DP, DP) for g in range(3)],
        axis=1).astype(jnp.bfloat16)
    dbhh = jnp.concatenate(
        [jnp.pad(dec_bhh[g * DH:(g + 1) * DH], (0, DP - DH)) for g in range(3)]
    )[None, :]
    fcwT = _pad2(fc_W.T, DP, OUTP)
    fcb = jnp.pad(fc_b, (0, OUTP - OUT))[None, :]

    g1 = pl.pallas_call(
        _prep_kernel,
        out_shape=jax.ShapeDtypeStruct((NP, NC), f32),
    )(emb_pad, ew, ev)

    gi = pl.pallas_call(
        _gemm_kernel,
        grid=(NCHUNK,),
        in_specs=[
            pl.BlockSpec((CHUNK, NP), lambda i: (i, 0)),
            pl.BlockSpec((NP, NC), lambda i: (0, 0)),
            pl.BlockSpec((1, NC), lambda i: (0, 0)),
            pl.BlockSpec((NC, G3), lambda i: (0, 0)),
            pl.BlockSpec((1, G3), lambda i: (0, 0)),
        ],
        out_specs=pl.BlockSpec((CHUNK, G3), lambda i: (i, 0)),
        out_shape=jax.ShapeDtypeStruct((BT, G3), f32),
    )(x_pad, g1, brow, wihT, bih)

    h_end = pl.pallas_call(
        _enc_kernel,
        grid=(NCHUNK,),
        in_specs=[
            pl.BlockSpec((TCH, B, G3), lambda i: (i, 0, 0)),
            pl.BlockSpec((H, G3), lambda i: (0, 0)),
            pl.BlockSpec((1, G3), lambda i: (0, 0)),
        ],
        out_specs=pl.BlockSpec((B, H), lambda i: (0, 0)),
        out_shape=jax.ShapeDtypeStruct((B, H), f32),
        scratch_shapes=[pltpu.VMEM((B, H), f32)],
    )(gi.reshape(T, B, G3), whhT, bhh)

    rep = jnp.repeat(h_end, T, axis=1).reshape(B, T, H)
    rep_tb = rep.transpose(1, 0, 2).reshape(BT, H)

    out = pl.pallas_call(
        _dec_kernel,
        in_specs=[
            pl.BlockSpec((BT, H), lambda: (0, 0)),
            pl.BlockSpec((H, G3D), lambda: (0, 0)),
            pl.BlockSpec((1, G3D), lambda: (0, 0)),
            pl.BlockSpec((DP, G3D), lambda: (0, 0)),
            pl.BlockSpec((1, G3D), lambda: (0, 0)),
            pl.BlockSpec((DP, OUTP), lambda: (0, 0)),
            pl.BlockSpec((1, OUTP), lambda: (0, 0)),
        ],
        out_specs=pl.BlockSpec((BT, OUTP), lambda: (0, 0)),
        out_shape=jax.ShapeDtypeStruct((BT, OUTP), f32),
        scratch_shapes=[pltpu.VMEM((T, B, G3D), f32), pltpu.VMEM((T, B, DP), f32)],
    )(rep_tb, dwihT, dbih, dwhhT, dbhh, fcwT, fcb)

    return out[:, :OUT].reshape(T, B, OUT).transpose(1, 0, 2)
